# baseline (speedup 1.0000x reference)
import functools

import jax
import jax.numpy as jnp
from jax import lax
from jax.experimental import pallas as pl
from jax.experimental.pallas import tpu as pltpu
from jax.experimental.pallas import tpu_sc as plsc

_N = 10000
_F = 128
_H = 16
_C = 40

_NC = 2
_NS = 16
_NW = _NC * _NS
_CHUNK = 128
_G = 16
_NPAD = 10240
_ROWS_PER_TILE = _NPAD // _NS


def _sc_mesh():
    return plsc.VectorSubcoreMesh(
        core_axis_name="c", subcore_axis_name="s",
        num_cores=_NC, num_subcores=_NS)


def _make_agg(n_chunks: int):
    @functools.partial(
        pl.kernel,
        out_type=jax.ShapeDtypeStruct((_NC, _NPAD, _H), jnp.float32),
        mesh=_sc_mesh(),
        compiler_params=pltpu.CompilerParams(use_tc_tiling_on_sc=False),
        scratch_types=[
            pltpu.VMEM((n_chunks, _CHUNK), jnp.int32),
            pltpu.VMEM((n_chunks, _CHUNK), jnp.int32),
            pltpu.VMEM((2, _G, _CHUNK, _H), jnp.float32),
            pltpu.VMEM_SHARED((_NPAD, _H), jnp.float32),
            pltpu.SemaphoreType.DMA,
            pltpu.SemaphoreType.DMA,
        ],
    )
    def agg(table_hbm, src_hbm, dst_hbm, out_hbm,
            src_v, dst_v, rows_v, acc_sh, gsem, ssem):
        c = lax.axis_index("c")
        s = lax.axis_index("s")
        wid = c * _NS + s
        n_groups = n_chunks // _G
        n_pairs = n_groups // 2

        def _zero(i, _):
            rows_v[0, 0, i] = jnp.zeros((_H,), jnp.float32)
            return ()
        lax.fori_loop(0, _CHUNK, _zero, ())
        row0 = s * _ROWS_PER_TILE
        for r in range(_ROWS_PER_TILE // _CHUNK):
            pltpu.sync_copy(rows_v.at[0, 0],
                            acc_sh.at[pl.ds(row0 + r * _CHUNK, _CHUNK)])

        pltpu.sync_copy(src_hbm.at[wid], src_v)
        pltpu.sync_copy(dst_hbm.at[wid], dst_v)
        plsc.subcore_barrier()

        def _fire_gathers(g, half):
            for b in range(_G):
                pltpu.async_copy(table_hbm.at[src_v.at[g * _G + b]],
                                 rows_v.at[half, b], gsem)

        def _drain_gathers(g, half):
            for b in range(_G):
                pltpu.make_async_copy(table_hbm.at[src_v.at[g * _G + b]],
                                      rows_v.at[half, b], gsem).wait()

        def _fire_scatters(g, half):
            for b in range(_G):
                pltpu.async_copy(rows_v.at[half, b],
                                 acc_sh.at[dst_v.at[g * _G + b]], ssem,
                                 add=True)

        def _drain_scatters(g, half):
            for b in range(_G):
                pltpu.make_async_copy(rows_v.at[half, b],
                                      acc_sh.at[dst_v.at[g * _G + b]],
                                      ssem).wait()

        _fire_gathers(0, 0)

        def _pair(p, _):
            g0 = 2 * p
            g1 = g0 + 1
            _drain_gathers(g0, 0)

            @pl.when(p > 0)
            def _():
                _drain_scatters(g0 - 1, 1)

            _fire_gathers(g1, 1)
            _fire_scatters(g0, 0)

            _drain_gathers(g1, 1)
            _drain_scatters(g0, 0)

            @pl.when(p + 1 < n_pairs)
            def _():
                _fire_gathers(g1 + 1, 0)

            _fire_scatters(g1, 1)
            return ()

        lax.fori_loop(0, n_pairs, _pair, ())
        _drain_scatters(n_groups - 1, 1)

        plsc.subcore_barrier()
        pltpu.sync_copy(acc_sh.at[pl.ds(row0, _ROWS_PER_TILE)],
                        out_hbm.at[c, pl.ds(row0, _ROWS_PER_TILE)])

    return agg


def _make_degree(n_chunks: int):
    @functools.partial(
        pl.kernel,
        out_type=jax.ShapeDtypeStruct((_NC, _NPAD, _H), jnp.float32),
        mesh=_sc_mesh(),
        compiler_params=pltpu.CompilerParams(use_tc_tiling_on_sc=False),
        scratch_types=[
            pltpu.VMEM((n_chunks, _CHUNK), jnp.int32),
            pltpu.VMEM((_CHUNK, _H), jnp.float32),
            pltpu.VMEM_SHARED((_NPAD, _H), jnp.float32),
            pltpu.SemaphoreType.DMA,
        ],
    )
    def degree(dst_hbm, out_hbm, dst_v, rows_v, acc_sh, ssem):
        c = lax.axis_index("c")
        s = lax.axis_index("s")
        wid = c * _NS + s

        def _zero(i, _):
            rows_v[i] = jnp.zeros((_H,), jnp.float32)
            return ()
        lax.fori_loop(0, _CHUNK, _zero, ())
        row0 = s * _ROWS_PER_TILE
        for r in range(_ROWS_PER_TILE // _CHUNK):
            pltpu.sync_copy(rows_v, acc_sh.at[pl.ds(row0 + r * _CHUNK, _CHUNK)])

        pltpu.sync_copy(dst_hbm.at[wid], dst_v)

        def _ones(i, _):
            rows_v[i] = jnp.ones((_H,), jnp.float32)
            return ()
        lax.fori_loop(0, _CHUNK, _ones, ())
        plsc.subcore_barrier()

        def _fire(j):
            pltpu.async_copy(rows_v, acc_sh.at[dst_v.at[j]], ssem, add=True)

        def _drain(j):
            pltpu.make_async_copy(rows_v, acc_sh.at[dst_v.at[j]], ssem).wait()

        for j in range(_G):
            _fire(j)

        def _step(j, _):
            _fire(j)
            _drain(j - _G)
            return ()
        lax.fori_loop(_G, n_chunks, _step, ())
        for j in range(_G):
            _drain(j)

        plsc.subcore_barrier()
        pltpu.sync_copy(acc_sh.at[pl.ds(row0, _ROWS_PER_TILE)],
                        out_hbm.at[c, pl.ds(row0, _ROWS_PER_TILE)])

    return degree


_BN = 1024


def _tc0_body(x_ref, w_ref, degp_ref, dinv_ref, t_ref):
    deg = degp_ref[0, :, 0:1] + degp_ref[1, :, 0:1] + 1.0
    dinv = lax.rsqrt(deg)
    dinv_ref[...] = dinv
    xw = jnp.dot(x_ref[...], w_ref[...], preferred_element_type=jnp.float32)
    t_ref[...] = xw * dinv


def _tc0(xp, W1, degp):
    grid = _NPAD // _BN
    return pl.pallas_call(
        _tc0_body,
        grid=(grid,),
        in_specs=[
            pl.BlockSpec((_BN, _F), lambda i: (i, 0)),
            pl.BlockSpec((_F, _H), lambda i: (0, 0)),
            pl.BlockSpec((_NC, _BN, _H), lambda i: (0, i, 0)),
        ],
        out_specs=[
            pl.BlockSpec((_BN, 1), lambda i: (i, 0)),
            pl.BlockSpec((_BN, _H), lambda i: (i, 0)),
        ],
        out_shape=[
            jax.ShapeDtypeStruct((_NPAD, 1), jnp.float32),
            jax.ShapeDtypeStruct((_NPAD, _H), jnp.float32),
        ],
    )(xp, W1, degp)


def _combine_body(p_ref, t_ref, dinv_ref, b_ref, wn_ref, tn_ref):
    dinv = dinv_ref[...]
    h = dinv * (p_ref[0] + p_ref[1] + t_ref[...]) + b_ref[...]
    h = jnp.maximum(h, 0.0)
    xw = jnp.dot(h, wn_ref[...], preferred_element_type=jnp.float32)
    tn_ref[...] = xw * dinv


def _combine(p, t, dinv, b, Wn):
    grid = _NPAD // _BN
    return pl.pallas_call(
        _combine_body,
        grid=(grid,),
        in_specs=[
            pl.BlockSpec((_NC, _BN, _H), lambda i: (0, i, 0)),
            pl.BlockSpec((_BN, _H), lambda i: (i, 0)),
            pl.BlockSpec((_BN, 1), lambda i: (i, 0)),
            pl.BlockSpec((1, _H), lambda i: (0, 0)),
            pl.BlockSpec((_H, _H), lambda i: (0, 0)),
        ],
        out_specs=pl.BlockSpec((_BN, _H), lambda i: (i, 0)),
        out_shape=jax.ShapeDtypeStruct((_NPAD, _H), jnp.float32),
    )(p, t, dinv, b, Wn)


def _final_body(p_ref, t_ref, dinv_ref, b_ref, w5_ref, out_ref):
    g = p_ref[0] + p_ref[1] + t_ref[...]
    logits = dinv_ref[...] * jnp.dot(
        g, w5_ref[...], preferred_element_type=jnp.float32) + b_ref[...]
    m = jnp.max(logits, axis=1, keepdims=True)
    z = logits - m
    lse = jnp.log(jnp.sum(jnp.exp(z), axis=1, keepdims=True))
    out_ref[...] = z - lse


def _final(p, t, dinv, b5, W5):
    grid = _NPAD // _BN
    return pl.pallas_call(
        _final_body,
        grid=(grid,),
        in_specs=[
            pl.BlockSpec((_NC, _BN, _H), lambda i: (0, i, 0)),
            pl.BlockSpec((_BN, _H), lambda i: (i, 0)),
            pl.BlockSpec((_BN, 1), lambda i: (i, 0)),
            pl.BlockSpec((1, _C), lambda i: (0, 0)),
            pl.BlockSpec((_H, _C), lambda i: (0, 0)),
        ],
        out_specs=pl.BlockSpec((_BN, _C), lambda i: (i, 0)),
        out_shape=jax.ShapeDtypeStruct((_NPAD, _C), jnp.float32),
    )(p, t, dinv, b5, W5)


def kernel(x, edge_index, W1, b1, W2, b2, W3, b3, W4, b4, W5, b5):
    E = edge_index.shape[1]
    n_chunks = -(-E // (_NW * _CHUNK))
    n_chunks = -(-n_chunks // (2 * _G)) * (2 * _G)
    per_tile = n_chunks * _CHUNK
    e_pad = per_tile * _NW

    src = jnp.full((e_pad,), _N, jnp.int32).at[:E].set(edge_index[0])
    dst = jnp.full((e_pad,), _N, jnp.int32).at[:E].set(edge_index[1])
    src_slab = src.reshape(_NW, n_chunks, _CHUNK)
    dst_slab = dst.reshape(_NW, n_chunks, _CHUNK)

    xp = jnp.zeros((_NPAD, _F), jnp.float32).at[:_N].set(x)

    agg = _make_agg(n_chunks)
    degp = _make_degree(n_chunks)(dst_slab)
    dinv, t = _tc0(xp, W1, degp)

    eye = jnp.eye(_H, dtype=jnp.float32)
    for b, Wn in ((b1, W2), (b2, W3), (b3, W4), (b4, eye)):
        p = agg(t, src_slab, dst_slab)
        t = _combine(p, t, dinv, b.reshape(1, _H), Wn)

    p = agg(t, src_slab, dst_slab)
    out = _final(p, t, dinv, b5.reshape(1, _C), W5)
    return out[:_N]


# speedup vs baseline: 6.0776x; 6.0776x over previous
import functools

import jax
import jax.numpy as jnp
from jax import lax
from jax.experimental import pallas as pl
from jax.experimental.pallas import tpu as pltpu
from jax.experimental.pallas import tpu_sc as plsc

_N = 10000
_F = 128
_H = 16
_C = 40

_NC = 2
_NS = 16
_NW = _NC * _NS
_CHUNK = 128
_G = 8
_NPAD = 10240
_ROWS_PER_TILE = _NPAD // _NS


def _sc_mesh():
    return plsc.VectorSubcoreMesh(
        core_axis_name="c", subcore_axis_name="s",
        num_cores=_NC, num_subcores=_NS)


def _make_agg(n_chunks: int):
    @functools.partial(
        pl.kernel,
        out_type=jax.ShapeDtypeStruct((_NC, _NPAD, _H), jnp.float32),
        mesh=_sc_mesh(),
        compiler_params=pltpu.CompilerParams(use_tc_tiling_on_sc=False),
        scratch_types=[
            pltpu.VMEM((n_chunks, _CHUNK), jnp.int32),
            pltpu.VMEM((n_chunks, _CHUNK), jnp.int32),
            pltpu.VMEM((2, _G, _CHUNK, _H), jnp.float32),
            pltpu.VMEM_SHARED((_NPAD, _H), jnp.float32),
            pltpu.VMEM_SHARED((_NPAD, _H), jnp.float32),
            pltpu.SemaphoreType.DMA,
            pltpu.SemaphoreType.DMA,
        ],
    )
    def agg(table_hbm, src_hbm, dst_hbm, out_hbm,
            src_v, dst_v, rows_v, acc_sh, table_sh, gsem, ssem):
        c = lax.axis_index("c")
        s = lax.axis_index("s")
        wid = c * _NS + s
        n_groups = n_chunks // _G
        n_pairs = n_groups // 2

        def _zero(i, _):
            rows_v[0, 0, i] = jnp.zeros((_H,), jnp.float32)
            return ()
        lax.fori_loop(0, _CHUNK, _zero, ())
        row0 = s * _ROWS_PER_TILE
        for r in range(_ROWS_PER_TILE // _CHUNK):
            pltpu.sync_copy(rows_v.at[0, 0],
                            acc_sh.at[pl.ds(row0 + r * _CHUNK, _CHUNK)])

        pltpu.sync_copy(src_hbm.at[wid], src_v)
        pltpu.sync_copy(dst_hbm.at[wid], dst_v)
        pltpu.sync_copy(table_hbm.at[pl.ds(row0, _ROWS_PER_TILE)],
                        table_sh.at[pl.ds(row0, _ROWS_PER_TILE)])
        plsc.subcore_barrier()

        def _fire_gathers(g, half):
            for b in range(_G):
                pltpu.async_copy(table_sh.at[src_v.at[g * _G + b]],
                                 rows_v.at[half, b], gsem)

        def _drain_gathers(g, half):
            for b in range(_G):
                pltpu.make_async_copy(table_sh.at[src_v.at[g * _G + b]],
                                      rows_v.at[half, b], gsem).wait()

        def _fire_scatters(g, half):
            for b in range(_G):
                pltpu.async_copy(rows_v.at[half, b],
                                 acc_sh.at[dst_v.at[g * _G + b]], ssem,
                                 add=True)

        def _drain_scatters(g, half):
            for b in range(_G):
                pltpu.make_async_copy(rows_v.at[half, b],
                                      acc_sh.at[dst_v.at[g * _G + b]],
                                      ssem).wait()

        _fire_gathers(0, 0)

        def _pair(p, _):
            g0 = 2 * p
            g1 = g0 + 1
            _drain_gathers(g0, 0)

            @pl.when(p > 0)
            def _():
                _drain_scatters(g0 - 1, 1)

            _fire_gathers(g1, 1)
            _fire_scatters(g0, 0)

            _drain_gathers(g1, 1)
            _drain_scatters(g0, 0)

            @pl.when(p + 1 < n_pairs)
            def _():
                _fire_gathers(g1 + 1, 0)

            _fire_scatters(g1, 1)
            return ()

        lax.fori_loop(0, n_pairs, _pair, ())
        _drain_scatters(n_groups - 1, 1)

        plsc.subcore_barrier()
        pltpu.sync_copy(acc_sh.at[pl.ds(row0, _ROWS_PER_TILE)],
                        out_hbm.at[c, pl.ds(row0, _ROWS_PER_TILE)])

    return agg


def _make_degree(n_chunks: int):
    @functools.partial(
        pl.kernel,
        out_type=jax.ShapeDtypeStruct((_NC, _NPAD, _H), jnp.float32),
        mesh=_sc_mesh(),
        compiler_params=pltpu.CompilerParams(use_tc_tiling_on_sc=False),
        scratch_types=[
            pltpu.VMEM((n_chunks, _CHUNK), jnp.int32),
            pltpu.VMEM((_CHUNK, _H), jnp.float32),
            pltpu.VMEM_SHARED((_NPAD, _H), jnp.float32),
            pltpu.SemaphoreType.DMA,
        ],
    )
    def degree(dst_hbm, out_hbm, dst_v, rows_v, acc_sh, ssem):
        c = lax.axis_index("c")
        s = lax.axis_index("s")
        wid = c * _NS + s

        def _zero(i, _):
            rows_v[i] = jnp.zeros((_H,), jnp.float32)
            return ()
        lax.fori_loop(0, _CHUNK, _zero, ())
        row0 = s * _ROWS_PER_TILE
        for r in range(_ROWS_PER_TILE // _CHUNK):
            pltpu.sync_copy(rows_v, acc_sh.at[pl.ds(row0 + r * _CHUNK, _CHUNK)])

        pltpu.sync_copy(dst_hbm.at[wid], dst_v)

        def _ones(i, _):
            rows_v[i] = jnp.ones((_H,), jnp.float32)
            return ()
        lax.fori_loop(0, _CHUNK, _ones, ())
        plsc.subcore_barrier()

        def _fire(j):
            pltpu.async_copy(rows_v, acc_sh.at[dst_v.at[j]], ssem, add=True)

        def _drain(j):
            pltpu.make_async_copy(rows_v, acc_sh.at[dst_v.at[j]], ssem).wait()

        for j in range(_G):
            _fire(j)

        def _step(j, _):
            _fire(j)
            _drain(j - _G)
            return ()
        lax.fori_loop(_G, n_chunks, _step, ())
        for j in range(_G):
            _drain(j)

        plsc.subcore_barrier()
        pltpu.sync_copy(acc_sh.at[pl.ds(row0, _ROWS_PER_TILE)],
                        out_hbm.at[c, pl.ds(row0, _ROWS_PER_TILE)])

    return degree


_BN = 1024


def _tc0_body(x_ref, w_ref, degp_ref, dinv_ref, t_ref):
    deg = degp_ref[0, :, 0:1] + degp_ref[1, :, 0:1] + 1.0
    dinv = lax.rsqrt(deg)
    dinv_ref[...] = dinv
    xw = jnp.dot(x_ref[...], w_ref[...], preferred_element_type=jnp.float32)
    t_ref[...] = xw * dinv


def _tc0(xp, W1, degp):
    grid = _NPAD // _BN
    return pl.pallas_call(
        _tc0_body,
        grid=(grid,),
        in_specs=[
            pl.BlockSpec((_BN, _F), lambda i: (i, 0)),
            pl.BlockSpec((_F, _H), lambda i: (0, 0)),
            pl.BlockSpec((_NC, _BN, _H), lambda i: (0, i, 0)),
        ],
        out_specs=[
            pl.BlockSpec((_BN, 1), lambda i: (i, 0)),
            pl.BlockSpec((_BN, _H), lambda i: (i, 0)),
        ],
        out_shape=[
            jax.ShapeDtypeStruct((_NPAD, 1), jnp.float32),
            jax.ShapeDtypeStruct((_NPAD, _H), jnp.float32),
        ],
    )(xp, W1, degp)


def _combine_body(p_ref, t_ref, dinv_ref, b_ref, wn_ref, tn_ref):
    dinv = dinv_ref[...]
    h = dinv * (p_ref[0] + p_ref[1] + t_ref[...]) + b_ref[...]
    h = jnp.maximum(h, 0.0)
    xw = jnp.dot(h, wn_ref[...], preferred_element_type=jnp.float32)
    tn_ref[...] = xw * dinv


def _combine(p, t, dinv, b, Wn):
    grid = _NPAD // _BN
    return pl.pallas_call(
        _combine_body,
        grid=(grid,),
        in_specs=[
            pl.BlockSpec((_NC, _BN, _H), lambda i: (0, i, 0)),
            pl.BlockSpec((_BN, _H), lambda i: (i, 0)),
            pl.BlockSpec((_BN, 1), lambda i: (i, 0)),
            pl.BlockSpec((1, _H), lambda i: (0, 0)),
            pl.BlockSpec((_H, _H), lambda i: (0, 0)),
        ],
        out_specs=pl.BlockSpec((_BN, _H), lambda i: (i, 0)),
        out_shape=jax.ShapeDtypeStruct((_NPAD, _H), jnp.float32),
    )(p, t, dinv, b, Wn)


def _final_body(p_ref, t_ref, dinv_ref, b_ref, w5_ref, out_ref):
    g = p_ref[0] + p_ref[1] + t_ref[...]
    logits = dinv_ref[...] * jnp.dot(
        g, w5_ref[...], preferred_element_type=jnp.float32) + b_ref[...]
    m = jnp.max(logits, axis=1, keepdims=True)
    z = logits - m
    lse = jnp.log(jnp.sum(jnp.exp(z), axis=1, keepdims=True))
    out_ref[...] = z - lse


def _final(p, t, dinv, b5, W5):
    grid = _NPAD // _BN
    return pl.pallas_call(
        _final_body,
        grid=(grid,),
        in_specs=[
            pl.BlockSpec((_NC, _BN, _H), lambda i: (0, i, 0)),
            pl.BlockSpec((_BN, _H), lambda i: (i, 0)),
            pl.BlockSpec((_BN, 1), lambda i: (i, 0)),
            pl.BlockSpec((1, _C), lambda i: (0, 0)),
            pl.BlockSpec((_H, _C), lambda i: (0, 0)),
        ],
        out_specs=pl.BlockSpec((_BN, _C), lambda i: (i, 0)),
        out_shape=jax.ShapeDtypeStruct((_NPAD, _C), jnp.float32),
    )(p, t, dinv, b5, W5)


def kernel(x, edge_index, W1, b1, W2, b2, W3, b3, W4, b4, W5, b5):
    E = edge_index.shape[1]
    n_chunks = -(-E // (_NW * _CHUNK))
    n_chunks = -(-n_chunks // (2 * _G)) * (2 * _G)
    per_tile = n_chunks * _CHUNK
    e_pad = per_tile * _NW

    src = jnp.full((e_pad,), _N, jnp.int32).at[:E].set(edge_index[0])
    dst = jnp.full((e_pad,), _N, jnp.int32).at[:E].set(edge_index[1])
    src_slab = src.reshape(_NW, n_chunks, _CHUNK)
    dst_slab = dst.reshape(_NW, n_chunks, _CHUNK)

    xp = jnp.zeros((_NPAD, _F), jnp.float32).at[:_N].set(x)

    agg = _make_agg(n_chunks)
    degp = _make_degree(n_chunks)(dst_slab)
    dinv, t = _tc0(xp, W1, degp)

    eye = jnp.eye(_H, dtype=jnp.float32)
    for b, Wn in ((b1, W2), (b2, W3), (b3, W4), (b4, eye)):
        p = agg(t, src_slab, dst_slab)
        t = _combine(p, t, dinv, b.reshape(1, _H), Wn)

    p = agg(t, src_slab, dst_slab)
    out = _final(p, t, dinv, b5.reshape(1, _C), W5)
    return out[:_N]


# speedup vs baseline: 8.5642x; 1.4091x over previous
import functools

import jax
import jax.numpy as jnp
from jax import lax
from jax.experimental import pallas as pl
from jax.experimental.pallas import tpu as pltpu
from jax.experimental.pallas import tpu_sc as plsc

_N = 10000
_F = 128
_H = 16
_C = 40

_NC = 2
_NS = 16
_NW = _NC * _NS
_CHUNK = 128
_G = 8
_NPAD = 10240
_ROWS_PER_TILE = _NPAD // _NS


def _sc_mesh():
    return plsc.VectorSubcoreMesh(
        core_axis_name="c", subcore_axis_name="s",
        num_cores=_NC, num_subcores=_NS)


def _make_agg(n_chunks: int):
    @functools.partial(
        pl.kernel,
        out_type=jax.ShapeDtypeStruct((_NC, _NPAD, _H), jnp.float32),
        mesh=_sc_mesh(),
        compiler_params=pltpu.CompilerParams(use_tc_tiling_on_sc=False),
        scratch_types=[
            pltpu.VMEM((n_chunks, _CHUNK), jnp.int32),
            pltpu.VMEM((n_chunks, _CHUNK), jnp.int32),
            pltpu.VMEM((2, _G, _CHUNK, _H), jnp.float32),
            pltpu.VMEM_SHARED((_NPAD, _H), jnp.float32),
            pltpu.VMEM_SHARED((_NPAD, _H), jnp.float32),
            pltpu.SemaphoreType.DMA,
            pltpu.SemaphoreType.DMA,
        ],
    )
    def agg(table_hbm, src_hbm, dst_hbm, out_hbm,
            src_v, dst_v, rows_v, acc_sh, table_sh, gsem, ssem):
        c = lax.axis_index("c")
        s = lax.axis_index("s")
        wid = c * _NS + s
        n_groups = n_chunks // _G
        n_pairs = n_groups // 2

        def _zero(i, _):
            rows_v[0, 0, i] = jnp.zeros((_H,), jnp.float32)
            return ()
        lax.fori_loop(0, _CHUNK, _zero, ())
        row0 = s * _ROWS_PER_TILE
        for r in range(_ROWS_PER_TILE // _CHUNK):
            pltpu.sync_copy(rows_v.at[0, 0],
                            acc_sh.at[pl.ds(row0 + r * _CHUNK, _CHUNK)])

        pltpu.sync_copy(src_hbm.at[wid], src_v)
        pltpu.sync_copy(dst_hbm.at[wid], dst_v)
        pltpu.sync_copy(table_hbm.at[pl.ds(row0, _ROWS_PER_TILE)],
                        table_sh.at[pl.ds(row0, _ROWS_PER_TILE)])
        plsc.subcore_barrier()

        def _fire_gathers(g, half):
            for b in range(_G):
                pltpu.async_copy(table_sh.at[src_v.at[g * _G + b]],
                                 rows_v.at[half, b], gsem)

        def _drain_gathers(g, half):
            for b in range(_G):
                pltpu.make_async_copy(table_sh.at[src_v.at[g * _G + b]],
                                      rows_v.at[half, b], gsem).wait()

        def _fire_scatters(g, half):
            for b in range(_G):
                pltpu.async_copy(rows_v.at[half, b],
                                 acc_sh.at[dst_v.at[g * _G + b]], ssem,
                                 add=True)

        def _drain_scatters(g, half):
            for b in range(_G):
                pltpu.make_async_copy(rows_v.at[half, b],
                                      acc_sh.at[dst_v.at[g * _G + b]],
                                      ssem).wait()

        _fire_gathers(0, 0)

        def _pair(p, _):
            g0 = 2 * p
            g1 = g0 + 1
            _drain_gathers(g0, 0)

            @pl.when(p > 0)
            def _():
                _drain_scatters(g0 - 1, 1)

            _fire_gathers(g1, 1)
            _fire_scatters(g0, 0)

            _drain_gathers(g1, 1)
            _drain_scatters(g0, 0)

            @pl.when(p + 1 < n_pairs)
            def _():
                _fire_gathers(g1 + 1, 0)

            _fire_scatters(g1, 1)
            return ()

        lax.fori_loop(0, n_pairs, _pair, ())
        _drain_scatters(n_groups - 1, 1)

        plsc.subcore_barrier()
        pltpu.sync_copy(acc_sh.at[pl.ds(row0, _ROWS_PER_TILE)],
                        out_hbm.at[c, pl.ds(row0, _ROWS_PER_TILE)])

    return agg


def _make_degree(n_chunks: int):
    @functools.partial(
        pl.kernel,
        out_type=jax.ShapeDtypeStruct((_NC, _NPAD, _H), jnp.float32),
        mesh=_sc_mesh(),
        compiler_params=pltpu.CompilerParams(use_tc_tiling_on_sc=False),
        scratch_types=[
            pltpu.VMEM((n_chunks, _CHUNK), jnp.int32),
            pltpu.VMEM((_CHUNK, _H), jnp.float32),
            pltpu.VMEM_SHARED((_NPAD, _H), jnp.float32),
            pltpu.SemaphoreType.DMA,
        ],
    )
    def degree(dst_hbm, out_hbm, dst_v, rows_v, acc_sh, ssem):
        c = lax.axis_index("c")
        s = lax.axis_index("s")
        wid = c * _NS + s

        def _zero(i, _):
            rows_v[i] = jnp.zeros((_H,), jnp.float32)
            return ()
        lax.fori_loop(0, _CHUNK, _zero, ())
        row0 = s * _ROWS_PER_TILE
        for r in range(_ROWS_PER_TILE // _CHUNK):
            pltpu.sync_copy(rows_v, acc_sh.at[pl.ds(row0 + r * _CHUNK, _CHUNK)])

        pltpu.sync_copy(dst_hbm.at[wid], dst_v)

        def _ones(i, _):
            rows_v[i] = jnp.ones((_H,), jnp.float32)
            return ()
        lax.fori_loop(0, _CHUNK, _ones, ())
        plsc.subcore_barrier()

        def _fire(j):
            pltpu.async_copy(rows_v, acc_sh.at[dst_v.at[j]], ssem, add=True)

        def _drain(j):
            pltpu.make_async_copy(rows_v, acc_sh.at[dst_v.at[j]], ssem).wait()

        for j in range(_G):
            _fire(j)

        def _step(j, _):
            _fire(j)
            _drain(j - _G)
            return ()
        lax.fori_loop(_G, n_chunks, _step, ())
        for j in range(_G):
            _drain(j)

        plsc.subcore_barrier()
        pltpu.sync_copy(acc_sh.at[pl.ds(row0, _ROWS_PER_TILE)],
                        out_hbm.at[c, pl.ds(row0, _ROWS_PER_TILE)])

    return degree


_RPK = _NPAD // 8
_BPK = 256


def _tc0_body(x_ref, w_ref, degp_ref, dinv_ref, t_ref):
    deg = degp_ref[0] + degp_ref[1] + 1.0
    dinv = lax.rsqrt(deg)
    dinv_ref[...] = dinv
    xw = jnp.dot(x_ref[...], w_ref[...], preferred_element_type=jnp.float32)
    t_ref[...] = xw * dinv


def _tc0(xr, W1big, degp):
    grid = _RPK // _BPK
    return pl.pallas_call(
        _tc0_body,
        grid=(grid,),
        in_specs=[
            pl.BlockSpec((_BPK, 8 * _F), lambda i: (i, 0)),
            pl.BlockSpec((8 * _F, 128), lambda i: (0, 0)),
            pl.BlockSpec((_NC, _BPK, 128), lambda i: (0, i, 0)),
        ],
        out_specs=[
            pl.BlockSpec((_BPK, 128), lambda i: (i, 0)),
            pl.BlockSpec((_BPK, 128), lambda i: (i, 0)),
        ],
        out_shape=[
            jax.ShapeDtypeStruct((_RPK, 128), jnp.float32),
            jax.ShapeDtypeStruct((_RPK, 128), jnp.float32),
        ],
    )(xr, W1big, degp)


def _combine_body(p_ref, t_ref, dinv_ref, b_ref, bd_ref, tn_ref):
    dinv = dinv_ref[...]
    h = dinv * (p_ref[0] + p_ref[1] + t_ref[...]) + b_ref[...]
    h = jnp.maximum(h, 0.0)
    xw = jnp.dot(h, bd_ref[...], preferred_element_type=jnp.float32)
    tn_ref[...] = xw * dinv


def _combine(p, t, dinv, b_tiled, BD):
    grid = _RPK // _BPK
    return pl.pallas_call(
        _combine_body,
        grid=(grid,),
        in_specs=[
            pl.BlockSpec((_NC, _BPK, 128), lambda i: (0, i, 0)),
            pl.BlockSpec((_BPK, 128), lambda i: (i, 0)),
            pl.BlockSpec((_BPK, 128), lambda i: (i, 0)),
            pl.BlockSpec((1, 128), lambda i: (0, 0)),
            pl.BlockSpec((128, 128), lambda i: (0, 0)),
        ],
        out_specs=pl.BlockSpec((_BPK, 128), lambda i: (i, 0)),
        out_shape=jax.ShapeDtypeStruct((_RPK, 128), jnp.float32),
    )(p, t, dinv, b_tiled, BD)


def _final_mm_body(p_ref, t_ref, dinv_ref, b_ref, bd_ref, out_ref):
    g = dinv_ref[...] * (p_ref[0] + p_ref[1] + t_ref[...])
    out_ref[...] = jnp.dot(
        g, bd_ref[...], preferred_element_type=jnp.float32) + b_ref[...]


def _final_mm(p, t, dinv, b5_tiled, BD5):
    grid = _RPK // _BPK
    return pl.pallas_call(
        _final_mm_body,
        grid=(grid,),
        in_specs=[
            pl.BlockSpec((_NC, _BPK, 128), lambda i: (0, i, 0)),
            pl.BlockSpec((_BPK, 128), lambda i: (i, 0)),
            pl.BlockSpec((_BPK, 128), lambda i: (i, 0)),
            pl.BlockSpec((1, 8 * _C), lambda i: (0, 0)),
            pl.BlockSpec((128, 8 * _C), lambda i: (0, 0)),
        ],
        out_specs=pl.BlockSpec((_BPK, 8 * _C), lambda i: (i, 0)),
        out_shape=jax.ShapeDtypeStruct((_RPK, 8 * _C), jnp.float32),
    )(p, t, dinv, b5_tiled, BD5)


_BLS = 2048


def _lsm_body(z_ref, out_ref):
    logits = z_ref[...]
    m = jnp.max(logits, axis=1, keepdims=True)
    z = logits - m
    lse = jnp.log(jnp.sum(jnp.exp(z), axis=1, keepdims=True))
    out_ref[...] = z - lse


def _log_softmax(logits):
    grid = _NPAD // _BLS
    return pl.pallas_call(
        _lsm_body,
        grid=(grid,),
        in_specs=[pl.BlockSpec((_BLS, _C), lambda i: (i, 0))],
        out_specs=pl.BlockSpec((_BLS, _C), lambda i: (i, 0)),
        out_shape=jax.ShapeDtypeStruct((_NPAD, _C), jnp.float32),
    )(logits)


def kernel(x, edge_index, W1, b1, W2, b2, W3, b3, W4, b4, W5, b5):
    E = edge_index.shape[1]
    n_chunks = -(-E // (_NW * _CHUNK))
    n_chunks = -(-n_chunks // (2 * _G)) * (2 * _G)
    per_tile = n_chunks * _CHUNK
    e_pad = per_tile * _NW

    src = jnp.full((e_pad,), _N, jnp.int32).at[:E].set(edge_index[0])
    dst = jnp.full((e_pad,), _N, jnp.int32).at[:E].set(edge_index[1])
    src_slab = src.reshape(_NW, n_chunks, _CHUNK)
    dst_slab = dst.reshape(_NW, n_chunks, _CHUNK)

    xp = jnp.zeros((_NPAD, _F), jnp.float32).at[:_N].set(x)
    xr = xp.reshape(_RPK, 8 * _F)
    eye8 = jnp.eye(8, dtype=jnp.float32)
    W1big = jnp.kron(eye8, W1)

    agg = _make_agg(n_chunks)
    degp = _make_degree(n_chunks)(dst_slab)
    dinv, t = _tc0(xr, W1big, degp.reshape(_NC, _RPK, 128))

    eyeH = jnp.eye(_H, dtype=jnp.float32)
    for b, Wn in ((b1, W2), (b2, W3), (b3, W4), (b4, eyeH)):
        p = agg(t.reshape(_NPAD, _H), src_slab, dst_slab)
        t = _combine(p.reshape(_NC, _RPK, 128), t, dinv,
                     jnp.tile(b, 8).reshape(1, 128), jnp.kron(eye8, Wn))

    p = agg(t.reshape(_NPAD, _H), src_slab, dst_slab)
    logits = _final_mm(p.reshape(_NC, _RPK, 128), t, dinv,
                       jnp.tile(b5, 8).reshape(1, 8 * _C), jnp.kron(eye8, W5))
    out = _log_softmax(logits.reshape(_NPAD, _C))
    return out[:_N]


# speedup vs baseline: 8.7766x; 1.0248x over previous
import functools

import jax
import jax.numpy as jnp
from jax import lax
from jax.experimental import pallas as pl
from jax.experimental.pallas import tpu as pltpu
from jax.experimental.pallas import tpu_sc as plsc

_N = 10000
_F = 128
_H = 16
_C = 40

_NC = 2
_NS = 16
_NW = _NC * _NS
_CHUNK = 128
_G = 8
_NPAD = 10240
_ROWS_PER_TILE = _NPAD // _NS


def _sc_mesh():
    return plsc.VectorSubcoreMesh(
        core_axis_name="c", subcore_axis_name="s",
        num_cores=_NC, num_subcores=_NS)


def _make_agg(n_chunks: int):
    @functools.partial(
        pl.kernel,
        out_type=jax.ShapeDtypeStruct((_NC, _NPAD, _H), jnp.float32),
        mesh=_sc_mesh(),
        compiler_params=pltpu.CompilerParams(use_tc_tiling_on_sc=False),
        scratch_types=[
            pltpu.VMEM((n_chunks, _CHUNK), jnp.int32),
            pltpu.VMEM((n_chunks, _CHUNK), jnp.int32),
            pltpu.VMEM((2, _G, _CHUNK, _H), jnp.float32),
            pltpu.VMEM_SHARED((_NPAD, _H), jnp.float32),
            pltpu.VMEM_SHARED((_NPAD, _H), jnp.float32),
            pltpu.SemaphoreType.DMA,
            pltpu.SemaphoreType.DMA,
        ],
    )
    def agg(table_hbm, src_hbm, dst_hbm, out_hbm,
            src_v, dst_v, rows_v, acc_sh, table_sh, gsem, ssem):
        c = lax.axis_index("c")
        s = lax.axis_index("s")
        wid = c * _NS + s
        n_groups = n_chunks // _G
        n_pairs = n_groups // 2

        def _zero(i, _):
            rows_v[0, 0, i] = jnp.zeros((_H,), jnp.float32)
            return ()
        lax.fori_loop(0, _CHUNK, _zero, ())
        row0 = s * _ROWS_PER_TILE
        for r in range(_ROWS_PER_TILE // _CHUNK):
            pltpu.sync_copy(rows_v.at[0, 0],
                            acc_sh.at[pl.ds(row0 + r * _CHUNK, _CHUNK)])

        pltpu.sync_copy(src_hbm.at[wid], src_v)
        pltpu.sync_copy(dst_hbm.at[wid], dst_v)
        pltpu.sync_copy(table_hbm.at[pl.ds(row0, _ROWS_PER_TILE)],
                        table_sh.at[pl.ds(row0, _ROWS_PER_TILE)])
        plsc.subcore_barrier()

        def _fire_gathers(g, half):
            for b in range(_G):
                pltpu.async_copy(table_sh.at[src_v.at[g * _G + b]],
                                 rows_v.at[half, b], gsem)

        def _drain_gathers(g, half):
            for b in range(_G):
                pltpu.make_async_copy(table_sh.at[src_v.at[g * _G + b]],
                                      rows_v.at[half, b], gsem).wait()

        def _fire_scatters(g, half):
            for b in range(_G):
                pltpu.async_copy(rows_v.at[half, b],
                                 acc_sh.at[dst_v.at[g * _G + b]], ssem,
                                 add=True)

        def _drain_scatters(g, half):
            for b in range(_G):
                pltpu.make_async_copy(rows_v.at[half, b],
                                      acc_sh.at[dst_v.at[g * _G + b]],
                                      ssem).wait()

        _fire_gathers(0, 0)

        def _pair(p, _):
            g0 = 2 * p
            g1 = g0 + 1
            _drain_gathers(g0, 0)

            @pl.when(p > 0)
            def _():
                _drain_scatters(g0 - 1, 1)

            _fire_gathers(g1, 1)
            _fire_scatters(g0, 0)

            _drain_gathers(g1, 1)
            _drain_scatters(g0, 0)

            @pl.when(p + 1 < n_pairs)
            def _():
                _fire_gathers(g1 + 1, 0)

            _fire_scatters(g1, 1)
            return ()

        lax.fori_loop(0, n_pairs, _pair, ())
        _drain_scatters(n_groups - 1, 1)

        plsc.subcore_barrier()
        pltpu.sync_copy(acc_sh.at[pl.ds(row0, _ROWS_PER_TILE)],
                        out_hbm.at[c, pl.ds(row0, _ROWS_PER_TILE)])

    return agg


def _make_degree(n_chunks: int):
    @functools.partial(
        pl.kernel,
        out_type=jax.ShapeDtypeStruct((_NC, _NPAD, _H), jnp.float32),
        mesh=_sc_mesh(),
        compiler_params=pltpu.CompilerParams(use_tc_tiling_on_sc=False),
        scratch_types=[
            pltpu.VMEM((n_chunks, _CHUNK), jnp.int32),
            pltpu.VMEM((_CHUNK, _H), jnp.float32),
            pltpu.VMEM_SHARED((_NPAD, _H), jnp.float32),
            pltpu.SemaphoreType.DMA,
        ],
    )
    def degree(dst_hbm, out_hbm, dst_v, rows_v, acc_sh, ssem):
        c = lax.axis_index("c")
        s = lax.axis_index("s")
        wid = c * _NS + s

        def _zero(i, _):
            rows_v[i] = jnp.zeros((_H,), jnp.float32)
            return ()
        lax.fori_loop(0, _CHUNK, _zero, ())
        row0 = s * _ROWS_PER_TILE
        for r in range(_ROWS_PER_TILE // _CHUNK):
            pltpu.sync_copy(rows_v, acc_sh.at[pl.ds(row0 + r * _CHUNK, _CHUNK)])

        pltpu.sync_copy(dst_hbm.at[wid], dst_v)

        def _ones(i, _):
            rows_v[i] = jnp.ones((_H,), jnp.float32)
            return ()
        lax.fori_loop(0, _CHUNK, _ones, ())
        plsc.subcore_barrier()

        def _fire(j):
            pltpu.async_copy(rows_v, acc_sh.at[dst_v.at[j]], ssem, add=True)

        def _drain(j):
            pltpu.make_async_copy(rows_v, acc_sh.at[dst_v.at[j]], ssem).wait()

        for j in range(_G):
            _fire(j)

        def _step(j, _):
            _fire(j)
            _drain(j - _G)
            return ()
        lax.fori_loop(_G, n_chunks, _step, ())
        for j in range(_G):
            _drain(j)

        plsc.subcore_barrier()
        pltpu.sync_copy(acc_sh.at[pl.ds(row0, _ROWS_PER_TILE)],
                        out_hbm.at[c, pl.ds(row0, _ROWS_PER_TILE)])

    return degree


_RPK = _NPAD // 8
_BPK = 256


def _mm0_body(x_ref, w_ref, xw_ref):
    xw_ref[...] = jnp.dot(
        x_ref[...], w_ref[...], preferred_element_type=jnp.float32)


def _mm0(xr, W1big):
    grid = _RPK // _BPK
    return pl.pallas_call(
        _mm0_body,
        grid=(grid,),
        in_specs=[
            pl.BlockSpec((_BPK, 8 * _F), lambda i: (i, 0)),
            pl.BlockSpec((8 * _F, 128), lambda i: (0, 0)),
        ],
        out_specs=pl.BlockSpec((_BPK, 128), lambda i: (i, 0)),
        out_shape=jax.ShapeDtypeStruct((_RPK, 128), jnp.float32),
    )(xr, W1big)


def _scale0_body(xw_ref, degp_ref, dinv_ref, t_ref):
    deg = degp_ref[0] + degp_ref[1] + 1.0
    dinv = lax.rsqrt(deg)
    dinv_ref[...] = dinv
    t_ref[...] = xw_ref[...] * dinv


def _scale0(xw, degp):
    grid = _RPK // _BPK
    return pl.pallas_call(
        _scale0_body,
        grid=(grid,),
        in_specs=[
            pl.BlockSpec((_BPK, 128), lambda i: (i, 0)),
            pl.BlockSpec((_NC, _BPK, 128), lambda i: (0, i, 0)),
        ],
        out_specs=[
            pl.BlockSpec((_BPK, 128), lambda i: (i, 0)),
            pl.BlockSpec((_BPK, 128), lambda i: (i, 0)),
        ],
        out_shape=[
            jax.ShapeDtypeStruct((_RPK, 128), jnp.float32),
            jax.ShapeDtypeStruct((_RPK, 128), jnp.float32),
        ],
    )(xw, degp)


def _combine_body(p_ref, t_ref, dinv_ref, b_ref, bd_ref, tn_ref):
    dinv = dinv_ref[...]
    h = dinv * (p_ref[0] + p_ref[1] + t_ref[...]) + b_ref[...]
    h = jnp.maximum(h, 0.0)
    xw = jnp.dot(h, bd_ref[...], preferred_element_type=jnp.float32)
    tn_ref[...] = xw * dinv


def _combine(p, t, dinv, b_tiled, BD):
    grid = _RPK // _BPK
    return pl.pallas_call(
        _combine_body,
        grid=(grid,),
        in_specs=[
            pl.BlockSpec((_NC, _BPK, 128), lambda i: (0, i, 0)),
            pl.BlockSpec((_BPK, 128), lambda i: (i, 0)),
            pl.BlockSpec((_BPK, 128), lambda i: (i, 0)),
            pl.BlockSpec((1, 128), lambda i: (0, 0)),
            pl.BlockSpec((128, 128), lambda i: (0, 0)),
        ],
        out_specs=pl.BlockSpec((_BPK, 128), lambda i: (i, 0)),
        out_shape=jax.ShapeDtypeStruct((_RPK, 128), jnp.float32),
    )(p, t, dinv, b_tiled, BD)


def _final_mm_body(p_ref, t_ref, dinv_ref, b_ref, bd_ref, out_ref):
    g = dinv_ref[...] * (p_ref[0] + p_ref[1] + t_ref[...])
    z = jnp.dot(g, bd_ref[...], preferred_element_type=jnp.float32) + b_ref[...]
    for j in range(8):
        seg = z[:, j * _C:(j + 1) * _C]
        m = jnp.max(seg, axis=1, keepdims=True)
        s = seg - m
        lse = jnp.log(jnp.sum(jnp.exp(s), axis=1, keepdims=True))
        out_ref[:, j * _C:(j + 1) * _C] = s - lse


def _final_mm(p, t, dinv, b5_tiled, BD5):
    grid = _RPK // _BPK
    return pl.pallas_call(
        _final_mm_body,
        grid=(grid,),
        in_specs=[
            pl.BlockSpec((_NC, _BPK, 128), lambda i: (0, i, 0)),
            pl.BlockSpec((_BPK, 128), lambda i: (i, 0)),
            pl.BlockSpec((_BPK, 128), lambda i: (i, 0)),
            pl.BlockSpec((1, 8 * _C), lambda i: (0, 0)),
            pl.BlockSpec((128, 8 * _C), lambda i: (0, 0)),
        ],
        out_specs=pl.BlockSpec((_BPK, 8 * _C), lambda i: (i, 0)),
        out_shape=jax.ShapeDtypeStruct((_RPK, 8 * _C), jnp.float32),
    )(p, t, dinv, b5_tiled, BD5)


def kernel(x, edge_index, W1, b1, W2, b2, W3, b3, W4, b4, W5, b5):
    E = edge_index.shape[1]
    n_chunks = -(-E // (_NW * _CHUNK))
    n_chunks = -(-n_chunks // (2 * _G)) * (2 * _G)
    per_tile = n_chunks * _CHUNK
    e_pad = per_tile * _NW

    src = jnp.full((e_pad,), _N, jnp.int32).at[:E].set(edge_index[0])
    dst = jnp.full((e_pad,), _N, jnp.int32).at[:E].set(edge_index[1])
    src_slab = src.reshape(_NW, n_chunks, _CHUNK)
    dst_slab = dst.reshape(_NW, n_chunks, _CHUNK)

    xp = jnp.zeros((_NPAD, _F), jnp.float32).at[:_N].set(x)
    xr = xp.reshape(_RPK, 8 * _F)
    eye8 = jnp.eye(8, dtype=jnp.float32)
    W1big = jnp.kron(eye8, W1)

    agg = _make_agg(n_chunks)
    degp = _make_degree(n_chunks)(dst_slab)
    xw = _mm0(xr, W1big)
    dinv, t = _scale0(xw, degp.reshape(_NC, _RPK, 128))

    eyeH = jnp.eye(_H, dtype=jnp.float32)
    for b, Wn in ((b1, W2), (b2, W3), (b3, W4), (b4, eyeH)):
        p = agg(t.reshape(_NPAD, _H), src_slab, dst_slab)
        t = _combine(p.reshape(_NC, _RPK, 128), t, dinv,
                     jnp.tile(b, 8).reshape(1, 128), jnp.kron(eye8, Wn))

    p = agg(t.reshape(_NPAD, _H), src_slab, dst_slab)
    out = _final_mm(p.reshape(_NC, _RPK, 128), t, dinv,
                    jnp.tile(b5, 8).reshape(1, 8 * _C), jnp.kron(eye8, W5))
    return out.reshape(_NPAD, _C)[:_N]


# speedup vs baseline: 9.3118x; 1.0610x over previous
import functools

import jax
import jax.numpy as jnp
from jax import lax
from jax.experimental import pallas as pl
from jax.experimental.pallas import tpu as pltpu
from jax.experimental.pallas import tpu_sc as plsc

_N = 10000
_F = 128
_H = 16
_C = 40

_NC = 2
_NS = 16
_NW = _NC * _NS
_CHUNK = 128
_G = 8
_NPAD = 10240
_ROWS_PER_TILE = _NPAD // _NS


def _sc_mesh():
    return plsc.VectorSubcoreMesh(
        core_axis_name="c", subcore_axis_name="s",
        num_cores=_NC, num_subcores=_NS)


def _make_agg(n_chunks: int):
    @functools.partial(
        pl.kernel,
        out_type=jax.ShapeDtypeStruct((_NC, _NPAD, _H), jnp.float32),
        mesh=_sc_mesh(),
        compiler_params=pltpu.CompilerParams(use_tc_tiling_on_sc=False),
        scratch_types=[
            pltpu.VMEM((n_chunks, _CHUNK), jnp.int32),
            pltpu.VMEM((n_chunks, _CHUNK), jnp.int32),
            pltpu.VMEM((2, _G, _CHUNK, _H), jnp.float32),
            pltpu.VMEM_SHARED((_NPAD, _H), jnp.float32),
            pltpu.VMEM_SHARED((_NPAD, _H), jnp.float32),
            pltpu.SemaphoreType.DMA,
            pltpu.SemaphoreType.DMA,
        ],
    )
    def agg(table_hbm, src_hbm, dst_hbm, out_hbm,
            src_v, dst_v, rows_v, acc_sh, table_sh, gsem, ssem):
        c = lax.axis_index("c")
        s = lax.axis_index("s")
        wid = c * _NS + s
        n_groups = n_chunks // _G
        n_pairs = n_groups // 2

        row0 = s * _ROWS_PER_TILE
        pltpu.async_copy(src_hbm.at[wid], src_v, gsem)
        pltpu.async_copy(dst_hbm.at[wid], dst_v, gsem)
        pltpu.async_copy(table_hbm.at[pl.ds(row0, _ROWS_PER_TILE)],
                         table_sh.at[pl.ds(row0, _ROWS_PER_TILE)], gsem)

        def _zero(i, _):
            rows_v[0, 0, i] = jnp.zeros((_H,), jnp.float32)
            return ()
        lax.fori_loop(0, _CHUNK, _zero, ())
        for r in range(_ROWS_PER_TILE // _CHUNK):
            pltpu.async_copy(rows_v.at[0, 0],
                             acc_sh.at[pl.ds(row0 + r * _CHUNK, _CHUNK)], ssem)

        pltpu.make_async_copy(src_hbm.at[wid], src_v, gsem).wait()
        pltpu.make_async_copy(dst_hbm.at[wid], dst_v, gsem).wait()
        pltpu.make_async_copy(table_hbm.at[pl.ds(row0, _ROWS_PER_TILE)],
                              table_sh.at[pl.ds(row0, _ROWS_PER_TILE)],
                              gsem).wait()
        for r in range(_ROWS_PER_TILE // _CHUNK):
            pltpu.make_async_copy(rows_v.at[0, 0],
                                  acc_sh.at[pl.ds(row0 + r * _CHUNK, _CHUNK)],
                                  ssem).wait()
        plsc.subcore_barrier()

        def _fire_gathers(g, half):
            for b in range(_G):
                pltpu.async_copy(table_sh.at[src_v.at[g * _G + b]],
                                 rows_v.at[half, b], gsem)

        def _drain_gathers(g, half):
            for b in range(_G):
                pltpu.make_async_copy(table_sh.at[src_v.at[g * _G + b]],
                                      rows_v.at[half, b], gsem).wait()

        def _fire_scatters(g, half):
            for b in range(_G):
                pltpu.async_copy(rows_v.at[half, b],
                                 acc_sh.at[dst_v.at[g * _G + b]], ssem,
                                 add=True)

        def _drain_scatters(g, half):
            for b in range(_G):
                pltpu.make_async_copy(rows_v.at[half, b],
                                      acc_sh.at[dst_v.at[g * _G + b]],
                                      ssem).wait()

        _fire_gathers(0, 0)

        def _pair(p, _):
            g0 = 2 * p
            g1 = g0 + 1
            _drain_gathers(g0, 0)

            @pl.when(p > 0)
            def _():
                _drain_scatters(g0 - 1, 1)

            _fire_gathers(g1, 1)
            _fire_scatters(g0, 0)

            _drain_gathers(g1, 1)
            _drain_scatters(g0, 0)

            @pl.when(p + 1 < n_pairs)
            def _():
                _fire_gathers(g1 + 1, 0)

            _fire_scatters(g1, 1)
            return ()

        lax.fori_loop(0, n_pairs, _pair, ())
        _drain_scatters(n_groups - 1, 1)

        plsc.subcore_barrier()
        pltpu.sync_copy(acc_sh.at[pl.ds(row0, _ROWS_PER_TILE)],
                        out_hbm.at[c, pl.ds(row0, _ROWS_PER_TILE)])

    return agg


def _make_degree(n_chunks: int):
    @functools.partial(
        pl.kernel,
        out_type=jax.ShapeDtypeStruct((_NC, _NPAD, _H), jnp.float32),
        mesh=_sc_mesh(),
        compiler_params=pltpu.CompilerParams(use_tc_tiling_on_sc=False),
        scratch_types=[
            pltpu.VMEM((n_chunks, _CHUNK), jnp.int32),
            pltpu.VMEM((_CHUNK, _H), jnp.float32),
            pltpu.VMEM_SHARED((_NPAD, _H), jnp.float32),
            pltpu.SemaphoreType.DMA,
        ],
    )
    def degree(dst_hbm, out_hbm, dst_v, rows_v, acc_sh, ssem):
        c = lax.axis_index("c")
        s = lax.axis_index("s")
        wid = c * _NS + s

        row0 = s * _ROWS_PER_TILE
        pltpu.async_copy(dst_hbm.at[wid], dst_v, ssem)

        def _zero(i, _):
            rows_v[i] = jnp.zeros((_H,), jnp.float32)
            return ()
        lax.fori_loop(0, _CHUNK, _zero, ())
        for r in range(_ROWS_PER_TILE // _CHUNK):
            pltpu.async_copy(rows_v, acc_sh.at[pl.ds(row0 + r * _CHUNK, _CHUNK)],
                             ssem)
        for r in range(_ROWS_PER_TILE // _CHUNK):
            pltpu.make_async_copy(rows_v,
                                  acc_sh.at[pl.ds(row0 + r * _CHUNK, _CHUNK)],
                                  ssem).wait()
        pltpu.make_async_copy(dst_hbm.at[wid], dst_v, ssem).wait()

        def _ones(i, _):
            rows_v[i] = jnp.ones((_H,), jnp.float32)
            return ()
        lax.fori_loop(0, _CHUNK, _ones, ())
        plsc.subcore_barrier()

        def _fire(j):
            pltpu.async_copy(rows_v, acc_sh.at[dst_v.at[j]], ssem, add=True)

        def _drain(j):
            pltpu.make_async_copy(rows_v, acc_sh.at[dst_v.at[j]], ssem).wait()

        for j in range(_G):
            _fire(j)

        def _step(j, _):
            _fire(j)
            _drain(j - _G)
            return ()
        lax.fori_loop(_G, n_chunks, _step, ())
        for j in range(_G):
            _drain(j)

        plsc.subcore_barrier()
        pltpu.sync_copy(acc_sh.at[pl.ds(row0, _ROWS_PER_TILE)],
                        out_hbm.at[c, pl.ds(row0, _ROWS_PER_TILE)])

    return degree


_RPK = _NPAD // 8
_BPK = 256


def _mm0_body(x_ref, w_ref, xw_ref):
    xw_ref[...] = jnp.dot(
        x_ref[...], w_ref[...], preferred_element_type=jnp.float32)


def _mm0(xr, W1big):
    grid = _RPK // _BPK
    return pl.pallas_call(
        _mm0_body,
        grid=(grid,),
        in_specs=[
            pl.BlockSpec((_BPK, 8 * _F), lambda i: (i, 0)),
            pl.BlockSpec((8 * _F, 128), lambda i: (0, 0)),
        ],
        out_specs=pl.BlockSpec((_BPK, 128), lambda i: (i, 0)),
        out_shape=jax.ShapeDtypeStruct((_RPK, 128), jnp.float32),
    )(xr, W1big)


def _scale0_body(xw_ref, degp_ref, dinv_ref, t_ref):
    deg = degp_ref[0] + degp_ref[1] + 1.0
    dinv = lax.rsqrt(deg)
    dinv_ref[...] = dinv
    t_ref[...] = xw_ref[...] * dinv


def _scale0(xw, degp):
    grid = _RPK // _BPK
    return pl.pallas_call(
        _scale0_body,
        grid=(grid,),
        in_specs=[
            pl.BlockSpec((_BPK, 128), lambda i: (i, 0)),
            pl.BlockSpec((_NC, _BPK, 128), lambda i: (0, i, 0)),
        ],
        out_specs=[
            pl.BlockSpec((_BPK, 128), lambda i: (i, 0)),
            pl.BlockSpec((_BPK, 128), lambda i: (i, 0)),
        ],
        out_shape=[
            jax.ShapeDtypeStruct((_RPK, 128), jnp.float32),
            jax.ShapeDtypeStruct((_RPK, 128), jnp.float32),
        ],
    )(xw, degp)


def _combine_body(p_ref, t_ref, dinv_ref, b_ref, bd_ref, tn_ref):
    dinv = dinv_ref[...]
    h = dinv * (p_ref[0] + p_ref[1] + t_ref[...]) + b_ref[...]
    h = jnp.maximum(h, 0.0)
    xw = jnp.dot(h, bd_ref[...], preferred_element_type=jnp.float32)
    tn_ref[...] = xw * dinv


def _combine(p, t, dinv, b_tiled, BD):
    grid = _RPK // _BPK
    return pl.pallas_call(
        _combine_body,
        grid=(grid,),
        in_specs=[
            pl.BlockSpec((_NC, _BPK, 128), lambda i: (0, i, 0)),
            pl.BlockSpec((_BPK, 128), lambda i: (i, 0)),
            pl.BlockSpec((_BPK, 128), lambda i: (i, 0)),
            pl.BlockSpec((1, 128), lambda i: (0, 0)),
            pl.BlockSpec((128, 128), lambda i: (0, 0)),
        ],
        out_specs=pl.BlockSpec((_BPK, 128), lambda i: (i, 0)),
        out_shape=jax.ShapeDtypeStruct((_RPK, 128), jnp.float32),
    )(p, t, dinv, b_tiled, BD)


def _final_mm_body(p_ref, t_ref, dinv_ref, b_ref, bd_ref, out_ref):
    g = dinv_ref[...] * (p_ref[0] + p_ref[1] + t_ref[...])
    z = jnp.dot(g, bd_ref[...], preferred_element_type=jnp.float32) + b_ref[...]
    for j in range(8):
        seg = z[:, j * _C:(j + 1) * _C]
        m = jnp.max(seg, axis=1, keepdims=True)
        s = seg - m
        lse = jnp.log(jnp.sum(jnp.exp(s), axis=1, keepdims=True))
        out_ref[:, j * _C:(j + 1) * _C] = s - lse


def _final_mm(p, t, dinv, b5_tiled, BD5):
    grid = _RPK // _BPK
    return pl.pallas_call(
        _final_mm_body,
        grid=(grid,),
        in_specs=[
            pl.BlockSpec((_NC, _BPK, 128), lambda i: (0, i, 0)),
            pl.BlockSpec((_BPK, 128), lambda i: (i, 0)),
            pl.BlockSpec((_BPK, 128), lambda i: (i, 0)),
            pl.BlockSpec((1, 8 * _C), lambda i: (0, 0)),
            pl.BlockSpec((128, 8 * _C), lambda i: (0, 0)),
        ],
        out_specs=pl.BlockSpec((_BPK, 8 * _C), lambda i: (i, 0)),
        out_shape=jax.ShapeDtypeStruct((_RPK, 8 * _C), jnp.float32),
    )(p, t, dinv, b5_tiled, BD5)


def kernel(x, edge_index, W1, b1, W2, b2, W3, b3, W4, b4, W5, b5):
    E = edge_index.shape[1]
    n_chunks = -(-E // (_NW * _CHUNK))
    n_chunks = -(-n_chunks // (2 * _G)) * (2 * _G)
    per_tile = n_chunks * _CHUNK
    e_pad = per_tile * _NW

    src = jnp.full((e_pad,), _N, jnp.int32).at[:E].set(edge_index[0])
    dst = jnp.full((e_pad,), _N, jnp.int32).at[:E].set(edge_index[1])
    src_slab = src.reshape(_NW, n_chunks, _CHUNK)
    dst_slab = dst.reshape(_NW, n_chunks, _CHUNK)

    xp = jnp.zeros((_NPAD, _F), jnp.float32).at[:_N].set(x)
    xr = xp.reshape(_RPK, 8 * _F)
    eye8 = jnp.eye(8, dtype=jnp.float32)
    W1big = jnp.kron(eye8, W1)

    agg = _make_agg(n_chunks)
    degp = _make_degree(n_chunks)(dst_slab)
    xw = _mm0(xr, W1big)
    dinv, t = _scale0(xw, degp.reshape(_NC, _RPK, 128))

    eyeH = jnp.eye(_H, dtype=jnp.float32)
    for b, Wn in ((b1, W2), (b2, W3), (b3, W4), (b4, eyeH)):
        p = agg(t.reshape(_NPAD, _H), src_slab, dst_slab)
        t = _combine(p.reshape(_NC, _RPK, 128), t, dinv,
                     jnp.tile(b, 8).reshape(1, 128), jnp.kron(eye8, Wn))

    p = agg(t.reshape(_NPAD, _H), src_slab, dst_slab)
    out = _final_mm(p.reshape(_NC, _RPK, 128), t, dinv,
                    jnp.tile(b5, 8).reshape(1, 8 * _C), jnp.kron(eye8, W5))
    return out.reshape(_NPAD, _C)[:_N]


# speedup vs baseline: 10.2359x; 1.0992x over previous
import functools

import jax
import jax.numpy as jnp
from jax import lax
from jax.experimental import pallas as pl
from jax.experimental.pallas import tpu as pltpu
from jax.experimental.pallas import tpu_sc as plsc

_N = 10000
_F = 128
_H = 16
_C = 40

_NC = 2
_NS = 16
_NW = _NC * _NS
_CHUNK = 128
_G = 8
_NPAD = 10240
_ROWS_PER_TILE = _NPAD // _NS


def _sc_mesh():
    return plsc.VectorSubcoreMesh(
        core_axis_name="c", subcore_axis_name="s",
        num_cores=_NC, num_subcores=_NS)


def _make_agg(n_chunks: int, ept: int):
    full = (ept // _CHUNK) * _CHUNK
    tail = n_chunks * _CHUNK - full

    @functools.partial(
        pl.kernel,
        out_type=jax.ShapeDtypeStruct((_NC, _NPAD, _H), jnp.float32),
        mesh=_sc_mesh(),
        compiler_params=pltpu.CompilerParams(use_tc_tiling_on_sc=False),
        scratch_types=[
            pltpu.VMEM((n_chunks * _CHUNK,), jnp.int32),
            pltpu.VMEM((n_chunks * _CHUNK,), jnp.int32),
            pltpu.VMEM((2, _G, _CHUNK, _H), jnp.float32),
            pltpu.VMEM_SHARED((_NPAD, _H), jnp.float32),
            pltpu.VMEM_SHARED((_NPAD, _H), jnp.float32),
            pltpu.SemaphoreType.DMA,
            pltpu.SemaphoreType.DMA,
        ],
    )
    def agg(table_hbm, edge_hbm, tsrc_hbm, tdst_hbm, out_hbm,
            src_v, dst_v, rows_v, acc_sh, table_sh, gsem, ssem):
        c = lax.axis_index("c")
        s = lax.axis_index("s")
        wid = c * _NS + s
        n_groups = n_chunks // _G
        n_pairs = n_groups // 2

        row0 = s * _ROWS_PER_TILE
        base = wid * ept
        pltpu.async_copy(edge_hbm.at[0, pl.ds(base, full)],
                         src_v.at[pl.ds(0, full)], gsem)
        pltpu.async_copy(edge_hbm.at[1, pl.ds(base, full)],
                         dst_v.at[pl.ds(0, full)], gsem)
        pltpu.async_copy(tsrc_hbm.at[wid], src_v.at[pl.ds(full, tail)], gsem)
        pltpu.async_copy(tdst_hbm.at[wid], dst_v.at[pl.ds(full, tail)], gsem)
        pltpu.async_copy(table_hbm.at[pl.ds(row0, _ROWS_PER_TILE)],
                         table_sh.at[pl.ds(row0, _ROWS_PER_TILE)], gsem)

        def _zero(i, _):
            rows_v[0, 0, i] = jnp.zeros((_H,), jnp.float32)
            return ()
        lax.fori_loop(0, _CHUNK, _zero, ())
        for r in range(_ROWS_PER_TILE // _CHUNK):
            pltpu.async_copy(rows_v.at[0, 0],
                             acc_sh.at[pl.ds(row0 + r * _CHUNK, _CHUNK)], ssem)

        pltpu.make_async_copy(edge_hbm.at[0, pl.ds(base, full)],
                              src_v.at[pl.ds(0, full)], gsem).wait()
        pltpu.make_async_copy(edge_hbm.at[1, pl.ds(base, full)],
                              dst_v.at[pl.ds(0, full)], gsem).wait()
        pltpu.make_async_copy(tsrc_hbm.at[wid],
                              src_v.at[pl.ds(full, tail)], gsem).wait()
        pltpu.make_async_copy(tdst_hbm.at[wid],
                              dst_v.at[pl.ds(full, tail)], gsem).wait()
        pltpu.make_async_copy(table_hbm.at[pl.ds(row0, _ROWS_PER_TILE)],
                              table_sh.at[pl.ds(row0, _ROWS_PER_TILE)],
                              gsem).wait()
        for r in range(_ROWS_PER_TILE // _CHUNK):
            pltpu.make_async_copy(rows_v.at[0, 0],
                                  acc_sh.at[pl.ds(row0 + r * _CHUNK, _CHUNK)],
                                  ssem).wait()
        plsc.subcore_barrier()

        def _idx(ref, j):
            return ref.at[pl.ds(j * _CHUNK, _CHUNK)]

        def _fire_gathers(g, half):
            for b in range(_G):
                pltpu.async_copy(table_sh.at[_idx(src_v, g * _G + b)],
                                 rows_v.at[half, b], gsem)

        def _drain_gathers(g, half):
            for b in range(_G):
                pltpu.make_async_copy(table_sh.at[_idx(src_v, g * _G + b)],
                                      rows_v.at[half, b], gsem).wait()

        def _fire_scatters(g, half):
            for b in range(_G):
                pltpu.async_copy(rows_v.at[half, b],
                                 acc_sh.at[_idx(dst_v, g * _G + b)], ssem,
                                 add=True)

        def _drain_scatters(g, half):
            for b in range(_G):
                pltpu.make_async_copy(rows_v.at[half, b],
                                      acc_sh.at[_idx(dst_v, g * _G + b)],
                                      ssem).wait()

        _fire_gathers(0, 0)

        def _pair(p, _):
            g0 = 2 * p
            g1 = g0 + 1
            _drain_gathers(g0, 0)

            @pl.when(p > 0)
            def _():
                _drain_scatters(g0 - 1, 1)

            _fire_gathers(g1, 1)
            _fire_scatters(g0, 0)

            _drain_gathers(g1, 1)
            _drain_scatters(g0, 0)

            @pl.when(p + 1 < n_pairs)
            def _():
                _fire_gathers(g1 + 1, 0)

            _fire_scatters(g1, 1)
            return ()

        lax.fori_loop(0, n_pairs, _pair, ())
        _drain_scatters(n_groups - 1, 1)

        plsc.subcore_barrier()
        pltpu.sync_copy(acc_sh.at[pl.ds(row0, _ROWS_PER_TILE)],
                        out_hbm.at[c, pl.ds(row0, _ROWS_PER_TILE)])

    return agg


def _make_degree(n_chunks: int, ept: int):
    full = (ept // _CHUNK) * _CHUNK
    tail = n_chunks * _CHUNK - full

    @functools.partial(
        pl.kernel,
        out_type=jax.ShapeDtypeStruct((_NC, _NPAD, _H), jnp.float32),
        mesh=_sc_mesh(),
        compiler_params=pltpu.CompilerParams(use_tc_tiling_on_sc=False),
        scratch_types=[
            pltpu.VMEM((n_chunks * _CHUNK,), jnp.int32),
            pltpu.VMEM((_CHUNK, _H), jnp.float32),
            pltpu.VMEM_SHARED((_NPAD, _H), jnp.float32),
            pltpu.SemaphoreType.DMA,
        ],
    )
    def degree(edge_hbm, tdst_hbm, out_hbm, dst_v, rows_v, acc_sh, ssem):
        c = lax.axis_index("c")
        s = lax.axis_index("s")
        wid = c * _NS + s

        row0 = s * _ROWS_PER_TILE
        base = wid * ept
        pltpu.async_copy(edge_hbm.at[1, pl.ds(base, full)],
                         dst_v.at[pl.ds(0, full)], ssem)
        pltpu.async_copy(tdst_hbm.at[wid], dst_v.at[pl.ds(full, tail)], ssem)

        def _zero(i, _):
            rows_v[i] = jnp.zeros((_H,), jnp.float32)
            return ()
        lax.fori_loop(0, _CHUNK, _zero, ())
        for r in range(_ROWS_PER_TILE // _CHUNK):
            pltpu.async_copy(rows_v, acc_sh.at[pl.ds(row0 + r * _CHUNK, _CHUNK)],
                             ssem)
        for r in range(_ROWS_PER_TILE // _CHUNK):
            pltpu.make_async_copy(rows_v,
                                  acc_sh.at[pl.ds(row0 + r * _CHUNK, _CHUNK)],
                                  ssem).wait()
        pltpu.make_async_copy(edge_hbm.at[1, pl.ds(base, full)],
                              dst_v.at[pl.ds(0, full)], ssem).wait()
        pltpu.make_async_copy(tdst_hbm.at[wid],
                              dst_v.at[pl.ds(full, tail)], ssem).wait()

        def _ones(i, _):
            rows_v[i] = jnp.ones((_H,), jnp.float32)
            return ()
        lax.fori_loop(0, _CHUNK, _ones, ())
        plsc.subcore_barrier()

        def _fire(j):
            pltpu.async_copy(rows_v, acc_sh.at[dst_v.at[pl.ds(j * _CHUNK, _CHUNK)]],
                             ssem, add=True)

        def _drain(j):
            pltpu.make_async_copy(rows_v,
                                  acc_sh.at[dst_v.at[pl.ds(j * _CHUNK, _CHUNK)]],
                                  ssem).wait()

        for j in range(_G):
            _fire(j)

        def _step(j, _):
            _fire(j)
            _drain(j - _G)
            return ()
        lax.fori_loop(_G, n_chunks, _step, ())
        for j in range(_G):
            _drain(j)

        plsc.subcore_barrier()
        pltpu.sync_copy(acc_sh.at[pl.ds(row0, _ROWS_PER_TILE)],
                        out_hbm.at[c, pl.ds(row0, _ROWS_PER_TILE)])

    return degree


_RPK = _NPAD // 8
_BPK = 256


def _mm0_body(x_ref, w_ref, xw_ref):
    xw_ref[...] = jnp.dot(
        x_ref[...], w_ref[...], preferred_element_type=jnp.float32)


def _mm0(xr, W1big):
    grid = _RPK // _BPK
    return pl.pallas_call(
        _mm0_body,
        grid=(grid,),
        in_specs=[
            pl.BlockSpec((_BPK, 8 * _F), lambda i: (i, 0)),
            pl.BlockSpec((8 * _F, 128), lambda i: (0, 0)),
        ],
        out_specs=pl.BlockSpec((_BPK, 128), lambda i: (i, 0)),
        out_shape=jax.ShapeDtypeStruct((_RPK, 128), jnp.float32),
    )(xr, W1big)


def _scale0_body(xw_ref, degp_ref, dinv_ref, t_ref):
    deg = degp_ref[0] + degp_ref[1] + 1.0
    dinv = lax.rsqrt(deg)
    dinv_ref[...] = dinv
    t_ref[...] = xw_ref[...] * dinv


def _scale0(xw, degp):
    grid = _RPK // _BPK
    return pl.pallas_call(
        _scale0_body,
        grid=(grid,),
        in_specs=[
            pl.BlockSpec((_BPK, 128), lambda i: (i, 0)),
            pl.BlockSpec((_NC, _BPK, 128), lambda i: (0, i, 0)),
        ],
        out_specs=[
            pl.BlockSpec((_BPK, 128), lambda i: (i, 0)),
            pl.BlockSpec((_BPK, 128), lambda i: (i, 0)),
        ],
        out_shape=[
            jax.ShapeDtypeStruct((_RPK, 128), jnp.float32),
            jax.ShapeDtypeStruct((_RPK, 128), jnp.float32),
        ],
    )(xw, degp)


def _combine_body(p_ref, t_ref, dinv_ref, b_ref, bd_ref, tn_ref):
    dinv = dinv_ref[...]
    h = dinv * (p_ref[0] + p_ref[1] + t_ref[...]) + b_ref[...]
    h = jnp.maximum(h, 0.0)
    xw = jnp.dot(h, bd_ref[...], preferred_element_type=jnp.float32)
    tn_ref[...] = xw * dinv


def _combine(p, t, dinv, b_tiled, BD):
    grid = _RPK // _BPK
    return pl.pallas_call(
        _combine_body,
        grid=(grid,),
        in_specs=[
            pl.BlockSpec((_NC, _BPK, 128), lambda i: (0, i, 0)),
            pl.BlockSpec((_BPK, 128), lambda i: (i, 0)),
            pl.BlockSpec((_BPK, 128), lambda i: (i, 0)),
            pl.BlockSpec((1, 128), lambda i: (0, 0)),
            pl.BlockSpec((128, 128), lambda i: (0, 0)),
        ],
        out_specs=pl.BlockSpec((_BPK, 128), lambda i: (i, 0)),
        out_shape=jax.ShapeDtypeStruct((_RPK, 128), jnp.float32),
    )(p, t, dinv, b_tiled, BD)


def _final_mm_body(p_ref, t_ref, dinv_ref, b_ref, bd_ref, out_ref):
    g = dinv_ref[...] * (p_ref[0] + p_ref[1] + t_ref[...])
    z = jnp.dot(g, bd_ref[...], preferred_element_type=jnp.float32) + b_ref[...]
    for j in range(8):
        seg = z[:, j * _C:(j + 1) * _C]
        m = jnp.max(seg, axis=1, keepdims=True)
        s = seg - m
        lse = jnp.log(jnp.sum(jnp.exp(s), axis=1, keepdims=True))
        out_ref[:, j * _C:(j + 1) * _C] = s - lse


def _final_mm(p, t, dinv, b5_tiled, BD5):
    grid = _RPK // _BPK
    return pl.pallas_call(
        _final_mm_body,
        grid=(grid,),
        in_specs=[
            pl.BlockSpec((_NC, _BPK, 128), lambda i: (0, i, 0)),
            pl.BlockSpec((_BPK, 128), lambda i: (i, 0)),
            pl.BlockSpec((_BPK, 128), lambda i: (i, 0)),
            pl.BlockSpec((1, 8 * _C), lambda i: (0, 0)),
            pl.BlockSpec((128, 8 * _C), lambda i: (0, 0)),
        ],
        out_specs=pl.BlockSpec((_BPK, 8 * _C), lambda i: (i, 0)),
        out_shape=jax.ShapeDtypeStruct((_RPK, 8 * _C), jnp.float32),
    )(p, t, dinv, b5_tiled, BD5)


def kernel(x, edge_index, W1, b1, W2, b2, W3, b3, W4, b4, W5, b5):
    E = edge_index.shape[1]
    ept = E // _NW
    n_chunks = -(-ept // _CHUNK)
    n_chunks = -(-n_chunks // (2 * _G)) * (2 * _G)
    full = (ept // _CHUNK) * _CHUNK
    rem = ept - full
    tail = n_chunks * _CHUNK - full

    edges = edge_index.astype(jnp.int32).reshape(2, _NW, ept)
    tails = jnp.full((2, _NW, tail), _N, jnp.int32)
    if rem:
        tails = tails.at[:, :, :rem].set(edges[:, :, full:])
    tsrc, tdst = tails[0], tails[1]

    xp = jnp.zeros((_NPAD, _F), jnp.float32).at[:_N].set(x)
    xr = xp.reshape(_RPK, 8 * _F)
    eye8 = jnp.eye(8, dtype=jnp.float32)
    W1big = jnp.kron(eye8, W1)

    agg = _make_agg(n_chunks, ept)
    degp = _make_degree(n_chunks, ept)(edge_index.astype(jnp.int32), tdst)
    xw = _mm0(xr, W1big)
    dinv, t = _scale0(xw, degp.reshape(_NC, _RPK, 128))

    ei32 = edge_index.astype(jnp.int32)
    eyeH = jnp.eye(_H, dtype=jnp.float32)
    for b, Wn in ((b1, W2), (b2, W3), (b3, W4), (b4, eyeH)):
        p = agg(t.reshape(_NPAD, _H), ei32, tsrc, tdst)
        t = _combine(p.reshape(_NC, _RPK, 128), t, dinv,
                     jnp.tile(b, 8).reshape(1, 128), jnp.kron(eye8, Wn))

    p = agg(t.reshape(_NPAD, _H), ei32, tsrc, tdst)
    out = _final_mm(p.reshape(_NC, _RPK, 128), t, dinv,
                    jnp.tile(b5, 8).reshape(1, 8 * _C), jnp.kron(eye8, W5))
    return out.reshape(_NPAD, _C)[:_N]
